# baseline (speedup 1.0000x reference)
import functools

import jax
import jax.numpy as jnp
from jax.experimental import pallas as pl
from jax.experimental.pallas import tpu as pltpu


def _pick_block(n, target=512):
    for bm in (256, 200, 128, 80, 64, 40, 32, 16, 8):
        if bm <= target and n % bm == 0:
            return bm
    return n


def _body(x_ref, wc1_ref, bc1_ref, wc2_ref, bc2_ref, adj_ref, o_ref,
          s_ref, t_ref, h_ref, *, bm, hdim, cdim):
    p = pl.program_id(0)
    j = pl.program_id(1)

    @pl.when(jnp.logical_and(p == 0, j == 0))
    def _():
        s_ref[...] = jnp.dot(x_ref[...], wc1_ref[...],
                             preferred_element_type=jnp.float32)

    @pl.when(p == 0)
    def _():
        y = jnp.dot(adj_ref[...], s_ref[...],
                    preferred_element_type=jnp.float32) + bc1_ref[...]
        y = jnp.maximum(y, 0.0)
        h_ref[pl.ds(j * bm, bm), :] = y[:, :hdim] * y[:, hdim:]

    @pl.when(jnp.logical_and(p == 1, j == 0))
    def _():
        t_ref[...] = jnp.dot(h_ref[...], wc2_ref[...],
                             preferred_element_type=jnp.float32)

    @pl.when(p == 1)
    def _():
        y = jnp.dot(adj_ref[...], t_ref[...],
                    preferred_element_type=jnp.float32) + bc2_ref[...]
        v = y[:, :cdim] * y[:, cdim:]
        m = jnp.max(v, axis=1, keepdims=True)
        e = jnp.exp(v - m)
        o_ref[...] = (v - m) - jnp.log(jnp.sum(e, axis=1, keepdims=True))


def kernel(x, adj, W1, b1, W1_1, b1_1, W2, b2, W2_1, b2_1):
    n, nfeat = x.shape
    nhid = W1.shape[1]
    nclass = W2.shape[1]
    bm = _pick_block(n)

    wc1 = jnp.concatenate([W1, W1_1], axis=1)
    bc1 = jnp.concatenate([b1, b1_1])[None, :]
    wc2 = jnp.concatenate([W2, W2_1], axis=1)
    bc2 = jnp.concatenate([b2, b2_1])[None, :]

    out = pl.pallas_call(
        functools.partial(_body, bm=bm, hdim=nhid, cdim=nclass),
        grid=(2, n // bm),
        in_specs=[
            pl.BlockSpec((n, nfeat), lambda p, j: (0, 0)),
            pl.BlockSpec((nfeat, 2 * nhid), lambda p, j: (0, 0)),
            pl.BlockSpec((1, 2 * nhid), lambda p, j: (0, 0)),
            pl.BlockSpec((nhid, 2 * nclass), lambda p, j: (0, 0)),
            pl.BlockSpec((1, 2 * nclass), lambda p, j: (0, 0)),
            pl.BlockSpec((bm, n), lambda p, j: (j, 0)),
        ],
        out_specs=pl.BlockSpec((bm, nclass), lambda p, j: (j, 0)),
        out_shape=jax.ShapeDtypeStruct((n, nclass), jnp.float32),
        scratch_shapes=[
            pltpu.VMEM((n, 2 * nhid), jnp.float32),
            pltpu.VMEM((n, 2 * nclass), jnp.float32),
            pltpu.VMEM((n, nhid), jnp.float32),
        ],
        compiler_params=pltpu.CompilerParams(
            dimension_semantics=("arbitrary", "arbitrary")),
    )(x, wc1, bc1, wc2, bc2, adj)

    return out


# speedup vs baseline: 1.0321x; 1.0321x over previous
import functools

import jax
import jax.numpy as jnp
from jax.experimental import pallas as pl
from jax.experimental.pallas import tpu as pltpu


def _pick_block(n, target=512):
    for bm in (1024, 1000, 512, 400, 256, 200, 128, 80, 64, 40, 32, 16, 8):
        if bm <= target and n % bm == 0:
            return bm
    return n


def _body(x_ref, wc1_ref, bc1_ref, wc2_ref, bc2_ref, adj_ref, o_ref,
          s_ref, t_ref, h_ref, *, bm, hdim, cdim):
    p = pl.program_id(0)
    j = pl.program_id(1)

    @pl.when(jnp.logical_and(p == 0, j == 0))
    def _():
        s_ref[...] = jnp.dot(x_ref[...], wc1_ref[...],
                             preferred_element_type=jnp.float32)

    @pl.when(p == 0)
    def _():
        y = jnp.dot(adj_ref[...], s_ref[...],
                    preferred_element_type=jnp.float32) + bc1_ref[...]
        y = jnp.maximum(y, 0.0)
        h_ref[pl.ds(j * bm, bm), :] = y[:, :hdim] * y[:, hdim:]

    @pl.when(jnp.logical_and(p == 1, j == 0))
    def _():
        t_ref[...] = jnp.dot(h_ref[...], wc2_ref[...],
                             preferred_element_type=jnp.float32)

    @pl.when(p == 1)
    def _():
        y = jnp.dot(adj_ref[...], t_ref[...],
                    preferred_element_type=jnp.float32) + bc2_ref[...]
        v = y[:, :cdim] * y[:, cdim:]
        m = jnp.max(v, axis=1, keepdims=True)
        e = jnp.exp(v - m)
        o_ref[...] = (v - m) - jnp.log(jnp.sum(e, axis=1, keepdims=True))


def kernel(x, adj, W1, b1, W1_1, b1_1, W2, b2, W2_1, b2_1):
    n, nfeat = x.shape
    nhid = W1.shape[1]
    nclass = W2.shape[1]
    bm = _pick_block(n)

    wc1 = jnp.concatenate([W1, W1_1], axis=1)
    bc1 = jnp.concatenate([b1, b1_1])[None, :]
    wc2 = jnp.concatenate([W2, W2_1], axis=1)
    bc2 = jnp.concatenate([b2, b2_1])[None, :]

    out = pl.pallas_call(
        functools.partial(_body, bm=bm, hdim=nhid, cdim=nclass),
        grid=(2, n // bm),
        in_specs=[
            pl.BlockSpec((n, nfeat), lambda p, j: (0, 0)),
            pl.BlockSpec((nfeat, 2 * nhid), lambda p, j: (0, 0)),
            pl.BlockSpec((1, 2 * nhid), lambda p, j: (0, 0)),
            pl.BlockSpec((nhid, 2 * nclass), lambda p, j: (0, 0)),
            pl.BlockSpec((1, 2 * nclass), lambda p, j: (0, 0)),
            pl.BlockSpec((bm, n), lambda p, j: (j, 0)),
        ],
        out_specs=pl.BlockSpec((bm, nclass), lambda p, j: (j, 0)),
        out_shape=jax.ShapeDtypeStruct((n, nclass), jnp.float32),
        scratch_shapes=[
            pltpu.VMEM((n, 2 * nhid), jnp.float32),
            pltpu.VMEM((n, 2 * nclass), jnp.float32),
            pltpu.VMEM((n, nhid), jnp.float32),
        ],
        compiler_params=pltpu.CompilerParams(
            dimension_semantics=("arbitrary", "arbitrary"),
            vmem_limit_bytes=120 * 1024 * 1024),
    )(x, wc1, bc1, wc2, bc2, adj)

    return out


# speedup vs baseline: 1.0378x; 1.0055x over previous
import functools

import jax
import jax.numpy as jnp
from jax.experimental import pallas as pl
from jax.experimental.pallas import tpu as pltpu


def _pick_block(n, target=512):
    for bm in (1024, 1000, 512, 400, 256, 200, 128, 80, 64, 40, 32, 16, 8):
        if bm <= target and n % bm == 0:
            return bm
    return n


def _body(x_ref, wc1_ref, bc1_ref, wc2_ref, bc2_ref, adj_ref, o_ref,
          s_ref, t_ref, h_ref, *, bm, hdim, cdim):
    p = pl.program_id(0)
    j = pl.program_id(1)

    @pl.when(jnp.logical_and(p == 0, j == 0))
    def _():
        s_ref[...] = jnp.dot(x_ref[...], wc1_ref[...],
                             preferred_element_type=jnp.float32)

    @pl.when(p == 0)
    def _():
        y = jnp.dot(adj_ref[...], s_ref[...],
                    preferred_element_type=jnp.float32) + bc1_ref[...]
        y = jnp.maximum(y, 0.0)
        h_ref[pl.ds(j * bm, bm), :] = y[:, :hdim] * y[:, hdim:]

    @pl.when(jnp.logical_and(p == 1, j == 0))
    def _():
        t_ref[...] = jnp.dot(h_ref[...], wc2_ref[...],
                             preferred_element_type=jnp.float32)


    @pl.when(p == 1)
    def _():
        y = jnp.dot(adj_ref[...], t_ref[...],
                    preferred_element_type=jnp.float32) + bc2_ref[...]
        v = y[:, :cdim] * y[:, cdim:]
        m = jnp.max(v, axis=1, keepdims=True)
        e = jnp.exp(v - m)
        o_ref[...] = (v - m) - jnp.log(jnp.sum(e, axis=1, keepdims=True))


def kernel(x, adj, W1, b1, W1_1, b1_1, W2, b2, W2_1, b2_1):
    n, nfeat = x.shape
    nhid = W1.shape[1]
    nclass = W2.shape[1]
    bm = _pick_block(n)
    nj = n // bm

    def adj_like_map(p, j):
        return (p * (nj - 1) + (1 - 2 * p) * j, 0)

    wc1 = jnp.concatenate([W1, W1_1], axis=1)
    bc1 = jnp.concatenate([b1, b1_1])[None, :]
    wc2 = jnp.concatenate([W2, W2_1], axis=1)
    bc2 = jnp.concatenate([b2, b2_1])[None, :]

    out = pl.pallas_call(
        functools.partial(_body, bm=bm, hdim=nhid, cdim=nclass),
        grid=(2, n // bm),
        in_specs=[
            pl.BlockSpec((n, nfeat), lambda p, j: (0, 0)),
            pl.BlockSpec((nfeat, 2 * nhid), lambda p, j: (0, 0)),
            pl.BlockSpec((1, 2 * nhid), lambda p, j: (0, 0)),
            pl.BlockSpec((nhid, 2 * nclass), lambda p, j: (0, 0)),
            pl.BlockSpec((1, 2 * nclass), lambda p, j: (0, 0)),
            pl.BlockSpec((bm, n), adj_like_map),
        ],
        out_specs=pl.BlockSpec((bm, nclass), adj_like_map),
        out_shape=jax.ShapeDtypeStruct((n, nclass), jnp.float32),
        scratch_shapes=[
            pltpu.VMEM((n, 2 * nhid), jnp.float32),
            pltpu.VMEM((n, 2 * nclass), jnp.float32),
            pltpu.VMEM((n, nhid), jnp.float32),
        ],
        compiler_params=pltpu.CompilerParams(
            dimension_semantics=("arbitrary", "arbitrary"),
            vmem_limit_bytes=120 * 1024 * 1024),
    )(x, wc1, bc1, wc2, bc2, adj)

    return out


# speedup vs baseline: 1.0426x; 1.0046x over previous
import functools

import jax
import jax.numpy as jnp
from jax.experimental import pallas as pl
from jax.experimental.pallas import tpu as pltpu


def _pick_block(n, target=512):
    for bm in (512, 400, 256, 200, 128, 80, 64, 40, 32, 16, 8):
        if bm <= target and n % bm == 0:
            return bm
    return n


def _body(x_ref, wc1_ref, bc1_ref, wc2_ref, bc2_ref, adj_ref, o_ref,
          s_ref, t_ref, h_ref, *, bm, nj, hdim, cdim):
    t = pl.program_id(0)

    @pl.when(t == 0)
    def _():
        s_ref[...] = jnp.dot(x_ref[...], wc1_ref[...],
                             preferred_element_type=jnp.float32)

    @pl.when(t < nj)
    def _():
        y = jnp.dot(adj_ref[...], s_ref[...],
                    preferred_element_type=jnp.float32) + bc1_ref[...]
        y = jnp.maximum(y, 0.0)
        h_ref[pl.ds(t * bm, bm), :] = y[:, :hdim] * y[:, hdim:]

    @pl.when(t == nj - 1)
    def _():
        t_ref[...] = jnp.dot(h_ref[...], wc2_ref[...],
                             preferred_element_type=jnp.float32)

    @pl.when(t >= nj - 1)
    def _():
        y = jnp.dot(adj_ref[...], t_ref[...],
                    preferred_element_type=jnp.float32) + bc2_ref[...]
        v = y[:, :cdim] * y[:, cdim:]
        m = jnp.max(v, axis=1, keepdims=True)
        e = jnp.exp(v - m)
        o_ref[...] = (v - m) - jnp.log(jnp.sum(e, axis=1, keepdims=True))


def kernel(x, adj, W1, b1, W1_1, b1_1, W2, b2, W2_1, b2_1):
    n, nfeat = x.shape
    nhid = W1.shape[1]
    nclass = W2.shape[1]
    bm = _pick_block(n)
    nj = n // bm

    wc1 = jnp.concatenate([W1, W1_1], axis=1)
    bc1 = jnp.concatenate([b1, b1_1])[None, :]
    wc2 = jnp.concatenate([W2, W2_1], axis=1)
    bc2 = jnp.concatenate([b2, b2_1])[None, :]

    out = pl.pallas_call(
        functools.partial(_body, bm=bm, nj=nj, hdim=nhid, cdim=nclass),
        grid=(2 * nj - 1,),
        in_specs=[
            pl.BlockSpec((n, nfeat), lambda t: (0, 0)),
            pl.BlockSpec((nfeat, 2 * nhid), lambda t: (0, 0)),
            pl.BlockSpec((1, 2 * nhid), lambda t: (0, 0)),
            pl.BlockSpec((nhid, 2 * nclass), lambda t: (0, 0)),
            pl.BlockSpec((1, 2 * nclass), lambda t: (0, 0)),
            pl.BlockSpec((bm, n), lambda t: (t % nj, 0)),
        ],
        out_specs=pl.BlockSpec((bm, nclass), lambda t: (t % nj, 0)),
        out_shape=jax.ShapeDtypeStruct((n, nclass), jnp.float32),
        scratch_shapes=[
            pltpu.VMEM((n, 2 * nhid), jnp.float32),
            pltpu.VMEM((n, 2 * nclass), jnp.float32),
            pltpu.VMEM((n, nhid), jnp.float32),
        ],
        compiler_params=pltpu.CompilerParams(
            dimension_semantics=("arbitrary",),
            vmem_limit_bytes=120 * 1024 * 1024),
    )(x, wc1, bc1, wc2, bc2, adj)

    return out


# speedup vs baseline: 1.0442x; 1.0015x over previous
import functools

import jax
import jax.numpy as jnp
from jax.experimental import pallas as pl
from jax.experimental.pallas import tpu as pltpu


def _pick_block(n, target=512):
    for bm in (512, 400, 256, 200, 128, 80, 64, 40, 32, 16, 8):
        if bm <= target and n % bm == 0:
            return bm
    return n


def _body(x_ref, adj_ref, w1_ref, b1_ref, w11_ref, b11_ref,
          w2_ref, b2_ref, w21_ref, b21_ref, o_ref,
          s_ref, t_ref, h_ref, *, bm, nj, hdim, cdim):
    t = pl.program_id(0)

    @pl.when(t == 0)
    def _():
        s_ref[:, :hdim] = jnp.dot(x_ref[...], w1_ref[...],
                                  preferred_element_type=jnp.float32)
        s_ref[:, hdim:] = jnp.dot(x_ref[...], w11_ref[...],
                                  preferred_element_type=jnp.float32)

    @pl.when(t < nj)
    def _():
        y = jnp.dot(adj_ref[...], s_ref[...],
                    preferred_element_type=jnp.float32)
        y1 = jnp.maximum(y[:, :hdim] + b1_ref[...], 0.0)
        y2 = jnp.maximum(y[:, hdim:] + b11_ref[...], 0.0)
        h_ref[pl.ds(t * bm, bm), :] = y1 * y2

    @pl.when(t == nj - 1)
    def _():
        t_ref[:, :cdim] = jnp.dot(h_ref[...], w2_ref[...],
                                  preferred_element_type=jnp.float32)
        t_ref[:, cdim:] = jnp.dot(h_ref[...], w21_ref[...],
                                  preferred_element_type=jnp.float32)

    @pl.when(t >= nj - 1)
    def _():
        y = jnp.dot(adj_ref[...], t_ref[...],
                    preferred_element_type=jnp.float32)
        v = (y[:, :cdim] + b2_ref[...]) * (y[:, cdim:] + b21_ref[...])
        m = jnp.max(v, axis=1, keepdims=True)
        e = jnp.exp(v - m)
        o_ref[...] = (v - m) - jnp.log(jnp.sum(e, axis=1, keepdims=True))


def kernel(x, adj, W1, b1, W1_1, b1_1, W2, b2, W2_1, b2_1):
    n, nfeat = x.shape
    nhid = W1.shape[1]
    nclass = W2.shape[1]
    bm = _pick_block(n)
    nj = n // bm

    def full(shape):
        return pl.BlockSpec(shape, lambda t: (0,) * len(shape))

    out = pl.pallas_call(
        functools.partial(_body, bm=bm, nj=nj, hdim=nhid, cdim=nclass),
        grid=(2 * nj - 1,),
        in_specs=[
            full((n, nfeat)),
            pl.BlockSpec((bm, n), lambda t: (t % nj, 0)),
            full((nfeat, nhid)),
            full((nhid,)),
            full((nfeat, nhid)),
            full((nhid,)),
            full((nhid, nclass)),
            full((nclass,)),
            full((nhid, nclass)),
            full((nclass,)),
        ],
        out_specs=pl.BlockSpec((bm, nclass), lambda t: (t % nj, 0)),
        out_shape=jax.ShapeDtypeStruct((n, nclass), jnp.float32),
        scratch_shapes=[
            pltpu.VMEM((n, 2 * nhid), jnp.float32),
            pltpu.VMEM((n, 2 * nclass), jnp.float32),
            pltpu.VMEM((n, nhid), jnp.float32),
        ],
        compiler_params=pltpu.CompilerParams(
            dimension_semantics=("arbitrary",),
            vmem_limit_bytes=120 * 1024 * 1024),
    )(x, adj, W1, b1, W1_1, b1_1, W2, b2, W2_1, b2_1)

    return out


# speedup vs baseline: 1.0465x; 1.0022x over previous
import functools

import jax
import jax.numpy as jnp
from jax.experimental import pallas as pl
from jax.experimental.pallas import tpu as pltpu


def _pick_block(n, target=512):
    for bm in (512, 400, 256, 200, 128, 80, 64, 40, 32, 16, 8):
        if bm <= target and n % bm == 0:
            return bm
    return n


def _body(x_ref, adj_ref, w1_ref, b1_ref, w11_ref, b11_ref,
          w2_ref, b2_ref, w21_ref, b21_ref, o_ref,
          s_ref, t_ref, h_ref, *, bm, nj, hdim, cdim):
    t = pl.program_id(0)

    @pl.when(t == 0)
    def _():
        s_ref[:, :hdim] = jnp.dot(x_ref[...], w1_ref[...],
                                  preferred_element_type=jnp.float32)
        s_ref[:, hdim:] = jnp.dot(x_ref[...], w11_ref[...],
                                  preferred_element_type=jnp.float32)

    @pl.when(t < nj)
    def _():
        y = jnp.dot(adj_ref[...], s_ref[...],
                    preferred_element_type=jnp.float32)
        y1 = jnp.maximum(y[:, :hdim] + b1_ref[...], 0.0)
        y2 = jnp.maximum(y[:, hdim:] + b11_ref[...], 0.0)
        h_ref[pl.ds(t * bm, bm), :] = y1 * y2

    @pl.when(t == nj - 1)
    def _():
        t_ref[:, :cdim] = jnp.dot(h_ref[...], w2_ref[...],
                                  preferred_element_type=jnp.float32)
        t_ref[:, cdim:] = jnp.dot(h_ref[...], w21_ref[...],
                                  preferred_element_type=jnp.float32)

    @pl.when(t >= nj - 1)
    def _():
        y = jnp.dot(adj_ref[...], t_ref[...],
                    preferred_element_type=jnp.float32)
        v = (y[:, :cdim] + b2_ref[...]) * (y[:, cdim:] + b21_ref[...])
        m = jnp.max(v, axis=1, keepdims=True)
        e = jnp.exp(v - m)
        o_ref[...] = (v - m) - jnp.log(jnp.sum(e, axis=1, keepdims=True))


def kernel(x, adj, W1, b1, W1_1, b1_1, W2, b2, W2_1, b2_1):
    n, nfeat = x.shape
    nhid = W1.shape[1]
    nclass = W2.shape[1]
    bm = _pick_block(n)
    nj = n // bm

    def full(shape):
        return pl.BlockSpec(shape, lambda t: (0,) * len(shape))

    b1 = b1[None, :]
    b1_1 = b1_1[None, :]
    b2 = b2[None, :]
    b2_1 = b2_1[None, :]

    out = pl.pallas_call(
        functools.partial(_body, bm=bm, nj=nj, hdim=nhid, cdim=nclass),
        grid=(2 * nj - 1,),
        in_specs=[
            full((n, nfeat)),
            pl.BlockSpec((bm, n), lambda t: (t % nj, 0)),
            full((nfeat, nhid)),
            full((1, nhid)),
            full((nfeat, nhid)),
            full((1, nhid)),
            full((nhid, nclass)),
            full((1, nclass)),
            full((nhid, nclass)),
            full((1, nclass)),
        ],
        out_specs=pl.BlockSpec((bm, nclass), lambda t: (t % nj, 0)),
        out_shape=jax.ShapeDtypeStruct((n, nclass), jnp.float32),
        scratch_shapes=[
            pltpu.VMEM((n, 2 * nhid), jnp.float32),
            pltpu.VMEM((n, 2 * nclass), jnp.float32),
            pltpu.VMEM((n, nhid), jnp.float32),
        ],
        compiler_params=pltpu.CompilerParams(
            dimension_semantics=("arbitrary",),
            vmem_limit_bytes=120 * 1024 * 1024),
    )(x, adj, W1, b1, W1_1, b1_1, W2, b2, W2_1, b2_1)

    return out
